# baseline (speedup 1.0000x reference)
import functools

import jax
import jax.numpy as jnp
from jax import lax
from jax.experimental import pallas as pl
from jax.experimental.pallas import tpu as pltpu
from jax.experimental.pallas import tpu_sc as plsc

_B = 16384
_T = 200
_K = 24
_KK = _K * _K
_NC = 2
_NS = 16
_NW = _NC * _NS
_BPW = _B // _NW
_G = _BPW // 16


def _accum_step(idx_ref, yp_ref, at_ref, laneoff):
    def grp(g, carry):
        vi = idx_ref[pl.ds(g * 16, 16)]
        base = laneoff + vi * _K
        for c in range(_K):
            ypv = yp_ref[c, pl.ds(g * 16, 16)]
            plsc.addupdate_scatter(at_ref, [base + c], ypv)
        return carry

    return lax.fori_loop(0, _G, grp, 0)


def _sc_body(yp_hbm, yt_hbm, bt_hbm, out_hbm,
             bt_v, at_v, i0_v, i1_v, y0_v, y1_v, acc_v, sem0, sem1):
    cid = lax.axis_index("c")
    sid = lax.axis_index("s")
    wid = sid * _NC + cid
    i0 = wid * _BPW

    pltpu.sync_copy(bt_hbm, bt_v)

    zero = jnp.zeros((16,), jnp.float32)

    def zero_tab(q, carry):
        at_v[pl.ds(q * 16, 16)] = zero
        return carry

    lax.fori_loop(0, 16 * _KK // 16, zero_tab, 0)

    idx_bufs = (i0_v, i1_v)
    yp_bufs = (y0_v, y1_v)
    sems = (sem0, sem1)

    def start(t, b):
        pltpu.async_copy(yt_hbm.at[t, pl.ds(i0, _BPW)], idx_bufs[b], sems[b])
        pltpu.async_copy(yp_hbm.at[t, :, pl.ds(i0, _BPW)], yp_bufs[b], sems[b])

    def wait(b):
        pltpu.make_async_copy(yt_hbm.at[0, pl.ds(0, _BPW)], idx_bufs[b], sems[b]).wait()
        pltpu.make_async_copy(yp_hbm.at[0, :, pl.ds(0, _BPW)], yp_bufs[b], sems[b]).wait()

    for b in range(2):
        start(b, b)

    laneoff = lax.iota(jnp.int32, 16) * _KK

    def super_body(k, carry):
        for b in range(2):
            t = 2 * k + b
            wait(b)
            _accum_step(idx_bufs[b], yp_bufs[b], at_v, laneoff)

            @pl.when(t + 2 < _T)
            def _():
                start(t + 2, b)
        return carry

    lax.fori_loop(0, _T // 2, super_body, 0)

    def contract(q, acc):
        btv = bt_v[pl.ds(q * 16, 16)]
        s = at_v[pl.ds(q * 16, 16)]
        for l in range(1, 16):
            s = s + at_v[pl.ds(l * _KK + q * 16, 16)]
        return acc + btv * s

    acc_v[...] = lax.fori_loop(0, _KK // 16, contract, zero)
    pltpu.sync_copy(acc_v, out_hbm.at[wid])


@functools.partial(
    pl.kernel,
    mesh=plsc.VectorSubcoreMesh(core_axis_name="c", subcore_axis_name="s"),
    out_type=jax.ShapeDtypeStruct((_NW, 16), jnp.float32),
    compiler_params=pltpu.CompilerParams(needs_layout_passes=False),
    scratch_types=[
        pltpu.VMEM((_KK,), jnp.float32),
        pltpu.VMEM((16 * _KK,), jnp.float32),
        pltpu.VMEM((_BPW,), jnp.int32),
        pltpu.VMEM((_BPW,), jnp.int32),
        pltpu.VMEM((_K, _BPW), jnp.float32),
        pltpu.VMEM((_K, _BPW), jnp.float32),
        pltpu.VMEM((16,), jnp.float32),
        pltpu.SemaphoreType.DMA,
        pltpu.SemaphoreType.DMA,
    ],
)
def _sc_kernel(yp_hbm, yt_hbm, bt_hbm, out_hbm, *scratch):
    _sc_body(yp_hbm, yt_hbm, bt_hbm, out_hbm, *scratch)


def kernel(y_true, y_pred, B):
    ypt = jnp.transpose(y_pred, (1, 2, 0))
    ytt = jnp.transpose(y_true.astype(jnp.int32), (1, 0))
    bt = jnp.transpose(B, (1, 0)).reshape(_KK)
    out = _sc_kernel(ypt, ytt, bt)
    return jnp.sum(out)


# speedup vs baseline: 2.3261x; 2.3261x over previous
import functools

import jax
import jax.numpy as jnp
from jax import lax
from jax.experimental import pallas as pl
from jax.experimental.pallas import tpu as pltpu
from jax.experimental.pallas import tpu_sc as plsc

_B = 16384
_T = 200
_K = 24
_NC = 2
_NS = 16
_NW = _NC * _NS
_BPW = _B // _NW
_G = _BPW // 16
_NACC = 4


def _compute_step(idx_ref, yp_ref, bt_ref, accs):
    @plsc.parallel_loop(0, _G, unroll=2, carry=accs)
    def accs(g, accs):
        vi = idx_ref[pl.ds(g * 16, 16)]
        bbase = vi * _K
        accs = list(accs)
        for c in range(_K):
            bv = plsc.load_gather(bt_ref, [bbase + c])
            ypv = yp_ref[c, pl.ds(g * 16, 16)]
            accs[c % _NACC] = accs[c % _NACC] + ypv * bv
        return tuple(accs)

    return accs


def _sc_body(yp_hbm, yt_hbm, bt_hbm, out_hbm,
             bt_v, i0_v, i1_v, y0_v, y1_v, acc_v, sem0, sem1):
    cid = lax.axis_index("c")
    sid = lax.axis_index("s")
    wid = sid * _NC + cid
    i0 = wid * _BPW

    pltpu.sync_copy(bt_hbm, bt_v)

    idx_bufs = (i0_v, i1_v)
    yp_bufs = (y0_v, y1_v)
    sems = (sem0, sem1)

    def start(t, b):
        pltpu.async_copy(yt_hbm.at[t, pl.ds(i0, _BPW)], idx_bufs[b], sems[b])
        pltpu.async_copy(yp_hbm.at[t, :, pl.ds(i0, _BPW)], yp_bufs[b], sems[b])

    def wait(b):
        pltpu.make_async_copy(yt_hbm.at[0, pl.ds(0, _BPW)], idx_bufs[b], sems[b]).wait()
        pltpu.make_async_copy(yp_hbm.at[0, :, pl.ds(0, _BPW)], yp_bufs[b], sems[b]).wait()

    for b in range(2):
        start(b, b)

    zero = jnp.zeros((16,), jnp.float32)
    accs = (zero, zero, zero, zero)

    def super_body(k, accs):
        for b in range(2):
            t = 2 * k + b
            wait(b)
            accs = _compute_step(idx_bufs[b], yp_bufs[b], bt_v, accs)

            @pl.when(t + 2 < _T)
            def _():
                start(t + 2, b)
        return accs

    accs = lax.fori_loop(0, _T // 2, super_body, accs)
    acc_v[...] = accs[0] + accs[1] + accs[2] + accs[3]
    pltpu.sync_copy(acc_v, out_hbm.at[wid])


@functools.partial(
    pl.kernel,
    mesh=plsc.VectorSubcoreMesh(core_axis_name="c", subcore_axis_name="s"),
    out_type=jax.ShapeDtypeStruct((_NW, 16), jnp.float32),
    compiler_params=pltpu.CompilerParams(needs_layout_passes=False),
    scratch_types=[
        pltpu.VMEM((_K * _K,), jnp.float32),
        pltpu.VMEM((_BPW,), jnp.int32),
        pltpu.VMEM((_BPW,), jnp.int32),
        pltpu.VMEM((_K, _BPW), jnp.float32),
        pltpu.VMEM((_K, _BPW), jnp.float32),
        pltpu.VMEM((16,), jnp.float32),
        pltpu.SemaphoreType.DMA,
        pltpu.SemaphoreType.DMA,
    ],
)
def _sc_kernel(yp_hbm, yt_hbm, bt_hbm, out_hbm, *scratch):
    _sc_body(yp_hbm, yt_hbm, bt_hbm, out_hbm, *scratch)


def kernel(y_true, y_pred, B):
    ypt = jnp.transpose(y_pred, (1, 2, 0))
    ytt = jnp.transpose(y_true.astype(jnp.int32), (1, 0))
    bt = jnp.transpose(B, (1, 0)).reshape(_K * _K)
    out = _sc_kernel(ypt, ytt, bt)
    return jnp.sum(out)
